# baseline (speedup 1.0000x reference)
import jax
import jax.numpy as jnp
from jax import lax
from jax.experimental import pallas as pl
from jax.experimental.pallas import tpu as pltpu
from jax.experimental.pallas import tpu_sc as plsc

VOCAB = 100000
D_MODEL = 64
MAX_SEQ_LEN = 200
BATCH = 4096

N_TOKENS = BATCH * MAX_SEQ_LEN
NUM_CORES = 2
NUM_SUBCORES = 16
NUM_WORKERS = NUM_CORES * NUM_SUBCORES
N_SLICES = 4
SLICE_TOKENS = N_TOKENS // N_SLICES
PER_WORKER = SLICE_TOKENS // NUM_WORKERS
CHUNK = 640
SUBGATHER = 128
N_SUB = CHUNK // SUBGATHER
N_CHUNKS = PER_WORKER // CHUNK
NBUF = 2


def _positional_encoding():
    even_i = jnp.arange(0, D_MODEL, 2).astype(jnp.float32)
    denominator = jnp.power(10000.0, even_i / D_MODEL)
    position = jnp.arange(0, MAX_SEQ_LEN).reshape(MAX_SEQ_LEN, 1).astype(jnp.float32)
    even_pe = jnp.sin(position / denominator)
    odd_pe = jnp.cos(position / denominator)
    pe = jnp.stack([even_pe, odd_pe], axis=2).reshape(MAX_SEQ_LEN, D_MODEL)
    return pe


def _sc_body(x_ref, table_ref, out_ref, idx_v, dest_v, gsem0, gsem1,
             osem0, osem1):
    gsem = (gsem0, gsem1)
    osem = (osem0, osem1)
    cid = lax.axis_index("c")
    sid = lax.axis_index("s")
    wid = sid * NUM_CORES + cid
    base = wid * PER_WORKER

    def out_copy(c, b):
        return pltpu.make_async_copy(
            dest_v.at[b], out_ref.at[pl.ds(base + c * CHUNK, CHUNK)],
            osem[b])

    def gather_copy(b, j):
        return pltpu.make_async_copy(
            table_ref.at[idx_v.at[b].at[pl.ds(j * SUBGATHER, SUBGATHER)]],
            dest_v.at[b].at[pl.ds(j * SUBGATHER, SUBGATHER)],
            gsem[b])

    def prep(c, b):
        pltpu.sync_copy(x_ref.at[pl.ds(base + c * CHUNK, CHUNK)],
                        idx_v.at[b])
        for j in range(N_SUB):
            gather_copy(b, j).start()

    def finish(c, b):
        for j in range(N_SUB):
            gather_copy(b, j).wait()
        out_copy(c, b).start()

    def slot(c, b):
        nb = 1 - b
        out_copy(c - 1, nb).wait()
        prep(c + 1, nb)
        finish(c, b)

    prep(0, 0)
    prep(1, 1)
    finish(0, 0)

    def step(i, carry):
        slot(2 * i + 1, 1)
        slot(2 * i + 2, 0)
        return carry

    lax.fori_loop(0, (N_CHUNKS - 2) // 2, step, 0)

    out_copy(N_CHUNKS - 2, 0).wait()
    finish(N_CHUNKS - 1, 1)
    out_copy(N_CHUNKS - 1, 1).wait()


@jax.jit
def kernel(x, table):
    pe = _positional_encoding()
    x_flat = x.reshape(N_TOKENS)

    mesh = plsc.VectorSubcoreMesh(
        core_axis_name="c", subcore_axis_name="s", num_cores=NUM_CORES
    )
    run = pl.kernel(
        _sc_body,
        out_type=jax.ShapeDtypeStruct((SLICE_TOKENS, D_MODEL), jnp.float32),
        mesh=mesh,
        compiler_params=pltpu.CompilerParams(use_tc_tiling_on_sc=False),
        scratch_types=[
            pltpu.VMEM((NBUF, CHUNK), jnp.int32),
            pltpu.VMEM((NBUF, CHUNK, D_MODEL), jnp.float32),
            pltpu.SemaphoreType.DMA,
            pltpu.SemaphoreType.DMA,
            pltpu.SemaphoreType.DMA,
            pltpu.SemaphoreType.DMA,
        ],
    )

    parts = []
    rows = BATCH // N_SLICES
    for k in range(N_SLICES):
        xk = lax.slice(x_flat, (k * SLICE_TOKENS,), ((k + 1) * SLICE_TOKENS,))
        gk = run(xk, table)
        parts.append(gk.reshape(rows, MAX_SEQ_LEN, D_MODEL) + pe[None, :, :])
    return jnp.concatenate(parts, axis=0)


# speedup vs baseline: 1.1423x; 1.1423x over previous
import jax
import jax.numpy as jnp
from jax import lax
from jax.experimental import pallas as pl
from jax.experimental.pallas import tpu as pltpu
from jax.experimental.pallas import tpu_sc as plsc

VOCAB = 100000
D_MODEL = 64
MAX_SEQ_LEN = 200
BATCH = 4096

N_TOKENS = BATCH * MAX_SEQ_LEN
NUM_CORES = 2
NUM_SUBCORES = 16
NUM_WORKERS = NUM_CORES * NUM_SUBCORES
PER_WORKER = N_TOKENS // NUM_WORKERS
CHUNK = 2 * MAX_SEQ_LEN
SUBGATHER = 80
N_SUB = CHUNK // SUBGATHER
N_CHUNKS = PER_WORKER // CHUNK
NBUF = 2


def _positional_encoding():
    even_i = jnp.arange(0, D_MODEL, 2).astype(jnp.float32)
    denominator = jnp.power(10000.0, even_i / D_MODEL)
    position = jnp.arange(0, MAX_SEQ_LEN).reshape(MAX_SEQ_LEN, 1).astype(jnp.float32)
    even_pe = jnp.sin(position / denominator)
    odd_pe = jnp.cos(position / denominator)
    pe = jnp.stack([even_pe, odd_pe], axis=2).reshape(MAX_SEQ_LEN, D_MODEL)
    return pe


def _sc_body(x_ref, table_ref, pe_ref, out_ref, idx_v, dest_v, pe_sh,
             gsem0, gsem1, osem0, osem1):
    gsem = (gsem0, gsem1)
    osem = (osem0, osem1)
    cid = lax.axis_index("c")
    sid = lax.axis_index("s")
    wid = sid * NUM_CORES + cid
    base = wid * PER_WORKER

    @pl.when(sid == 0)
    def _():
        pltpu.sync_copy(pe_ref, pe_sh)

    plsc.subcore_barrier()

    def out_copy(c, b):
        return pltpu.make_async_copy(
            dest_v.at[b], out_ref.at[pl.ds(base + c * CHUNK, CHUNK)],
            osem[b])

    def gather_copy(b, j):
        return pltpu.make_async_copy(
            table_ref.at[idx_v.at[b].at[pl.ds(j * SUBGATHER, SUBGATHER)]],
            dest_v.at[b].at[pl.ds(j * SUBGATHER, SUBGATHER)],
            gsem[b])

    def prep(c, b):
        pltpu.sync_copy(x_ref.at[pl.ds(base + c * CHUNK, CHUNK)],
                        idx_v.at[b])
        pltpu.sync_copy(pe_sh, dest_v.at[b])
        for j in range(N_SUB):
            gather_copy(b, j).start(add=True)

    def finish(c, b):
        for j in range(N_SUB):
            gather_copy(b, j).wait()
        out_copy(c, b).start()

    def slot(c, b):
        nb = 1 - b
        out_copy(c - 1, nb).wait()
        prep(c + 1, nb)
        finish(c, b)

    prep(0, 0)
    prep(1, 1)
    finish(0, 0)

    def step(i, carry):
        slot(2 * i + 1, 1)
        slot(2 * i + 2, 0)
        return carry

    lax.fori_loop(0, (N_CHUNKS - 2) // 2, step, 0)

    out_copy(N_CHUNKS - 2, 0).wait()
    finish(N_CHUNKS - 1, 1)
    out_copy(N_CHUNKS - 1, 1).wait()


@jax.jit
def kernel(x, table):
    pe = _positional_encoding()
    pe_img = jnp.concatenate([pe, pe], axis=0)
    x_flat = x.reshape(N_TOKENS)

    mesh = plsc.VectorSubcoreMesh(
        core_axis_name="c", subcore_axis_name="s", num_cores=NUM_CORES
    )
    run = pl.kernel(
        _sc_body,
        out_type=jax.ShapeDtypeStruct((N_TOKENS, D_MODEL), jnp.float32),
        mesh=mesh,
        compiler_params=pltpu.CompilerParams(use_tc_tiling_on_sc=False),
        scratch_types=[
            pltpu.VMEM((NBUF, CHUNK), jnp.int32),
            pltpu.VMEM((NBUF, CHUNK, D_MODEL), jnp.float32),
            pltpu.VMEM_SHARED((CHUNK, D_MODEL), jnp.float32),
            pltpu.SemaphoreType.DMA,
            pltpu.SemaphoreType.DMA,
            pltpu.SemaphoreType.DMA,
            pltpu.SemaphoreType.DMA,
        ],
    )
    out = run(x_flat, table, pe_img)
    return out.reshape(BATCH, MAX_SEQ_LEN, D_MODEL)


# speedup vs baseline: 1.1885x; 1.0404x over previous
import jax
import jax.numpy as jnp
from jax import lax
from jax.experimental import pallas as pl
from jax.experimental.pallas import tpu as pltpu
from jax.experimental.pallas import tpu_sc as plsc

VOCAB = 100000
D_MODEL = 64
MAX_SEQ_LEN = 200
BATCH = 4096

N_TOKENS = BATCH * MAX_SEQ_LEN
NUM_CORES = 2
NUM_SUBCORES = 16
NUM_WORKERS = NUM_CORES * NUM_SUBCORES
PER_WORKER = N_TOKENS // NUM_WORKERS
CHUNK = 640
SUBGATHER = 128
N_SUB = CHUNK // SUBGATHER
N_CHUNKS = PER_WORKER // CHUNK
NBUF = 2
PE_ROWS = 160 + CHUNK


def _positional_encoding():
    even_i = jnp.arange(0, D_MODEL, 2).astype(jnp.float32)
    denominator = jnp.power(10000.0, even_i / D_MODEL)
    position = jnp.arange(0, MAX_SEQ_LEN).reshape(MAX_SEQ_LEN, 1).astype(jnp.float32)
    even_pe = jnp.sin(position / denominator)
    odd_pe = jnp.cos(position / denominator)
    pe = jnp.stack([even_pe, odd_pe], axis=2).reshape(MAX_SEQ_LEN, D_MODEL)
    return pe


def _sc_body(x_ref, table_ref, pe_ref, out_ref, idx_v, dest_v, pe_sh,
             gsem0, gsem1, osem0, osem1):
    gsem = (gsem0, gsem1)
    osem = (osem0, osem1)
    cid = lax.axis_index("c")
    sid = lax.axis_index("s")
    wid = sid * NUM_CORES + cid
    base = wid * PER_WORKER

    @pl.when(sid == 0)
    def _():
        pltpu.sync_copy(pe_ref, pe_sh)

    plsc.subcore_barrier()

    def out_copy(c, b):
        return pltpu.make_async_copy(
            dest_v.at[b], out_ref.at[pl.ds(base + c * CHUNK, CHUNK)],
            osem[b])

    def gather_copy(b, j):
        return pltpu.make_async_copy(
            table_ref.at[idx_v.at[b].at[pl.ds(j * SUBGATHER, SUBGATHER)]],
            dest_v.at[b].at[pl.ds(j * SUBGATHER, SUBGATHER)],
            gsem[b])

    def prep(c, b):
        pe_off = lax.rem(c * CHUNK, MAX_SEQ_LEN)
        pltpu.sync_copy(x_ref.at[pl.ds(base + c * CHUNK, CHUNK)],
                        idx_v.at[b])
        pltpu.sync_copy(pe_sh.at[pl.ds(pe_off, CHUNK)], dest_v.at[b])
        for j in range(N_SUB):
            gather_copy(b, j).start(add=True)

    def finish(c, b):
        for j in range(N_SUB):
            gather_copy(b, j).wait()
        out_copy(c, b).start()

    def slot(c, b):
        nb = 1 - b
        out_copy(c - 1, nb).wait()
        prep(c + 1, nb)
        finish(c, b)

    prep(0, 0)
    prep(1, 1)
    finish(0, 0)

    def step(i, carry):
        slot(2 * i + 1, 1)
        slot(2 * i + 2, 0)
        return carry

    lax.fori_loop(0, (N_CHUNKS - 2) // 2, step, 0)

    out_copy(N_CHUNKS - 2, 0).wait()
    finish(N_CHUNKS - 1, 1)
    out_copy(N_CHUNKS - 1, 1).wait()


@jax.jit
def kernel(x, table):
    pe = _positional_encoding()
    reps = (PE_ROWS + MAX_SEQ_LEN - 1) // MAX_SEQ_LEN
    pe_img = jnp.concatenate([pe] * reps, axis=0)[:PE_ROWS]
    x_flat = x.reshape(N_TOKENS)

    mesh = plsc.VectorSubcoreMesh(
        core_axis_name="c", subcore_axis_name="s", num_cores=NUM_CORES
    )
    run = pl.kernel(
        _sc_body,
        out_type=jax.ShapeDtypeStruct((N_TOKENS, D_MODEL), jnp.float32),
        mesh=mesh,
        compiler_params=pltpu.CompilerParams(use_tc_tiling_on_sc=False),
        scratch_types=[
            pltpu.VMEM((NBUF, CHUNK), jnp.int32),
            pltpu.VMEM((NBUF, CHUNK, D_MODEL), jnp.float32),
            pltpu.VMEM_SHARED((PE_ROWS, D_MODEL), jnp.float32),
            pltpu.SemaphoreType.DMA,
            pltpu.SemaphoreType.DMA,
            pltpu.SemaphoreType.DMA,
            pltpu.SemaphoreType.DMA,
        ],
    )
    out = run(x_flat, table, pe_img)
    return out.reshape(BATCH, MAX_SEQ_LEN, D_MODEL)
